# Initial kernel scaffold; baseline (speedup 1.0000x reference)
import jax
import jax.numpy as jnp
from jax.experimental import pallas as pl


def kernel(x, edge_index, W1, b1, W2, b2, W3, b3, Wh, bh):
    raise NotImplementedError("write your pallas kernel here")


# speedup vs baseline: 11.7584x; 11.7584x over previous
import functools

import jax
import jax.numpy as jnp
from jax import lax
from jax.experimental import pallas as pl
from jax.experimental.pallas import tpu as pltpu
from jax.experimental.pallas import tpu_sc as plsc

N = 10000
E = 320000
D = 128

NC, NS = 2, 16
NW = NC * NS
K = 80
NCHUNK = E // (NW * K)
S0 = 624
TAIL = N - NS * S0
ZR = 16
DW = 16

_mesh = plsc.VectorSubcoreMesh(core_axis_name="c", subcore_axis_name="s")


@functools.partial(
    pl.kernel,
    out_type=jax.ShapeDtypeStruct((NC * N, D), jnp.float32),
    mesh=_mesh,
    scratch_types=[
        pltpu.VMEM((NCHUNK, K), jnp.int32),
        pltpu.VMEM((NCHUNK, K), jnp.int32),
        pltpu.VMEM((K, D), jnp.float32),
        pltpu.VMEM((ZR, D), jnp.float32),
        pltpu.VMEM_SHARED((N, D), jnp.float32),
        pltpu.SemaphoreType.DMA,
    ],
)
def _scatter_rows(m_hbm, row_hbm, col_hbm, out_hbm, ridx, cidx, gbuf, zbuf,
                  acc, sem):
    c = lax.axis_index("c")
    s = lax.axis_index("s")
    w = c * NS + s
    base = pl.multiple_of(s * S0, 8)

    z = jnp.zeros((16,), jnp.float32)

    def _zrow(j, carry):
        for i in range(D // 16):
            zbuf[j, pl.ds(i * 16, 16)] = z
        return carry

    lax.fori_loop(0, ZR, _zrow, 0)
    for t in range(S0 // ZR):
        pltpu.sync_copy(zbuf, acc.at[pl.ds(base + t * ZR, ZR)])

    @pl.when(s == 0)
    def _zero_tail():
        pltpu.sync_copy(zbuf.at[pl.ds(0, TAIL)], acc.at[pl.ds(NS * S0, TAIL)])

    plsc.subcore_barrier()

    pltpu.sync_copy(row_hbm.at[w], ridx)
    pltpu.sync_copy(col_hbm.at[w], cidx)

    def _chunk(j, carry):
        pltpu.async_copy(m_hbm.at[ridx.at[j]], gbuf, sem).wait()
        pltpu.sync_copy(gbuf, acc.at[cidx.at[j]], add=True)
        return carry

    lax.fori_loop(0, NCHUNK, _chunk, 0)
    plsc.subcore_barrier()

    off = pl.multiple_of(c * N + s * S0, 8)
    pltpu.sync_copy(acc.at[pl.ds(base, S0)], out_hbm.at[pl.ds(off, S0)])

    @pl.when(s == 0)
    def _write_tail():
        toff = pl.multiple_of(c * N + NS * S0, 8)
        pltpu.sync_copy(acc.at[pl.ds(NS * S0, TAIL)],
                        out_hbm.at[pl.ds(toff, TAIL)])


@functools.partial(
    pl.kernel,
    out_type=jax.ShapeDtypeStruct((NC * N, D), jnp.float32),
    mesh=_mesh,
    scratch_types=[
        pltpu.VMEM((NCHUNK, K), jnp.int32),
        pltpu.VMEM((K, D), jnp.float32),
        pltpu.VMEM((ZR, D), jnp.float32),
        pltpu.VMEM_SHARED((N, D), jnp.float32),
    ],
)
def _degree(col_hbm, out_hbm, cidx, ones_v, zbuf, acc):
    c = lax.axis_index("c")
    s = lax.axis_index("s")
    w = c * NS + s
    base = pl.multiple_of(s * S0, 8)

    z = jnp.zeros((16,), jnp.float32)
    o = jnp.ones((16,), jnp.float32)

    def _zrow(j, carry):
        for i in range(D // 16):
            zbuf[j, pl.ds(i * 16, 16)] = z
        return carry

    lax.fori_loop(0, ZR, _zrow, 0)

    def _orow(j, carry):
        for i in range(D // 16):
            ones_v[j, pl.ds(i * 16, 16)] = o
        return carry

    lax.fori_loop(0, K, _orow, 0)
    for t in range(S0 // ZR):
        pltpu.sync_copy(zbuf, acc.at[pl.ds(base + t * ZR, ZR)])

    @pl.when(s == 0)
    def _zero_tail():
        pltpu.sync_copy(zbuf.at[pl.ds(0, TAIL)], acc.at[pl.ds(NS * S0, TAIL)])

    plsc.subcore_barrier()

    pltpu.sync_copy(col_hbm.at[w], cidx)

    def _chunk(j, carry):
        pltpu.sync_copy(ones_v, acc.at[cidx.at[j]], add=True)
        return carry

    lax.fori_loop(0, NCHUNK, _chunk, 0)
    plsc.subcore_barrier()

    off = pl.multiple_of(c * N + s * S0, 8)
    pltpu.sync_copy(acc.at[pl.ds(base, S0)], out_hbm.at[pl.ds(off, S0)])

    @pl.when(s == 0)
    def _write_tail():
        toff = pl.multiple_of(c * N + NS * S0, 8)
        pltpu.sync_copy(acc.at[pl.ds(NS * S0, TAIL)],
                        out_hbm.at[pl.ds(toff, TAIL)])


_R = 2000


def _mm_first_body(dg_ref, x_ref, w_ref, b_ref, dinv_ref, m_ref):
    deg = dg_ref[0] + dg_ref[1]
    dinv = jnp.where(deg > 0, lax.rsqrt(deg), 0.0)
    dinv_ref[...] = dinv[:, 0:DW]
    t = jnp.dot(x_ref[...], w_ref[...], preferred_element_type=jnp.float32)
    m_ref[...] = dinv[:, 0:1] * (t + b_ref[...])


def _mm_mid_body(dinv_ref, h_ref, sp_ref, w_ref, b_ref, hn_ref, m_ref):
    dinv = dinv_ref[:, 0:1]
    hn = jnp.maximum(dinv * (sp_ref[0] + sp_ref[1]) + h_ref[...], 0.0)
    hn_ref[...] = hn
    t = jnp.dot(hn, w_ref[...], preferred_element_type=jnp.float32)
    m_ref[...] = dinv * (t + b_ref[...])


def _mm_last_body(dinv_ref, h_ref, sp_ref, w_ref, b_ref, out_ref):
    dinv = dinv_ref[:, 0:1]
    hn = jnp.maximum(dinv * (sp_ref[0] + sp_ref[1]) + h_ref[...], 0.0)
    out_ref[...] = (
        jnp.dot(hn, w_ref[...], preferred_element_type=jnp.float32)
        + b_ref[...])


def _mm_first(degp, x, w, b):
    return pl.pallas_call(
        _mm_first_body,
        grid=(N // _R,),
        in_specs=[
            pl.BlockSpec((2, _R, D), lambda i: (0, i, 0)),
            pl.BlockSpec((_R, D), lambda i: (i, 0)),
            pl.BlockSpec((D, D), lambda i: (0, 0)),
            pl.BlockSpec((1, D), lambda i: (0, 0)),
        ],
        out_specs=[
            pl.BlockSpec((_R, DW), lambda i: (i, 0)),
            pl.BlockSpec((_R, D), lambda i: (i, 0)),
        ],
        out_shape=[
            jax.ShapeDtypeStruct((N, DW), jnp.float32),
            jax.ShapeDtypeStruct((N, D), jnp.float32),
        ],
    )(degp, x, w, b)


def _mm_mid(dinv, h, sp, w, b):
    return pl.pallas_call(
        _mm_mid_body,
        grid=(N // _R,),
        in_specs=[
            pl.BlockSpec((_R, DW), lambda i: (i, 0)),
            pl.BlockSpec((_R, D), lambda i: (i, 0)),
            pl.BlockSpec((2, _R, D), lambda i: (0, i, 0)),
            pl.BlockSpec((D, D), lambda i: (0, 0)),
            pl.BlockSpec((1, D), lambda i: (0, 0)),
        ],
        out_specs=[
            pl.BlockSpec((_R, D), lambda i: (i, 0)),
            pl.BlockSpec((_R, D), lambda i: (i, 0)),
        ],
        out_shape=[
            jax.ShapeDtypeStruct((N, D), jnp.float32),
            jax.ShapeDtypeStruct((N, D), jnp.float32),
        ],
    )(dinv, h, sp, w, b)


def _mm_last(dinv, h, sp, w, b):
    return pl.pallas_call(
        _mm_last_body,
        grid=(N // _R,),
        in_specs=[
            pl.BlockSpec((_R, DW), lambda i: (i, 0)),
            pl.BlockSpec((_R, D), lambda i: (i, 0)),
            pl.BlockSpec((2, _R, D), lambda i: (0, i, 0)),
            pl.BlockSpec((D, D), lambda i: (0, 0)),
            pl.BlockSpec((1, D), lambda i: (0, 0)),
        ],
        out_specs=pl.BlockSpec((_R, D), lambda i: (i, 0)),
        out_shape=jax.ShapeDtypeStruct((N, D), jnp.float32),
    )(dinv, h, sp, w, b)


def kernel(x, edge_index, W1, b1, W2, b2, W3, b3, Wh, bh):
    row = edge_index[0].reshape(NW, NCHUNK, K)
    col = edge_index[1].reshape(NW, NCHUNK, K)
    b1r = b1.reshape(1, D)
    b2r = b2.reshape(1, D)
    b3r = b3.reshape(1, D)
    bhr = bh.reshape(1, D)

    degp = _degree(col).reshape(NC, N, D)
    dinv, m = _mm_first(degp, x, W1, b1r)
    h = x
    for w, b in ((W2, b2r), (W3, b3r)):
        sp = _scatter_rows(m, row, col).reshape(NC, N, D)
        h, m = _mm_mid(dinv, h, sp, w, b)
    sp = _scatter_rows(m, row, col).reshape(NC, N, D)
    return _mm_last(dinv, h, sp, Wh, bhr)


# speedup vs baseline: 15.5063x; 1.3187x over previous
import functools

import jax
import jax.numpy as jnp
from jax import lax
from jax.experimental import pallas as pl
from jax.experimental.pallas import tpu as pltpu
from jax.experimental.pallas import tpu_sc as plsc

N = 10000
E = 320000
D = 128

NC, NS = 2, 16
NW = NC * NS
K = 80
NCHUNK = E // (NW * K)
KC = 128
CH = 78
NTAIL = (E - NW * CH * KC) // KC
S0 = 624
TAIL = N - NS * S0
ZR = 16
DW = 16

_mesh = plsc.VectorSubcoreMesh(core_axis_name="c", subcore_axis_name="s")


@functools.partial(
    pl.kernel,
    out_type=jax.ShapeDtypeStruct((NC * N, D), jnp.float32),
    mesh=_mesh,
    scratch_types=[
        pltpu.VMEM((1, KC), jnp.int32),
        pltpu.VMEM((1, KC), jnp.int32),
        pltpu.VMEM((1, KC), jnp.int32),
        pltpu.VMEM((1, KC), jnp.int32),
        pltpu.VMEM((KC, D), jnp.float32),
        pltpu.VMEM((KC, D), jnp.float32),
        pltpu.VMEM((ZR, D), jnp.float32),
        pltpu.VMEM_SHARED((N, D), jnp.float32),
        pltpu.SemaphoreType.DMA,
        pltpu.SemaphoreType.DMA,
        pltpu.SemaphoreType.DMA,
        pltpu.SemaphoreType.DMA,
        pltpu.SemaphoreType.DMA,
        pltpu.SemaphoreType.DMA,
        pltpu.SemaphoreType.DMA,
        pltpu.SemaphoreType.DMA,
    ],
)
def _scatter_rows(m_hbm, row_hbm, col_hbm, trow_hbm, tcol_hbm, out_hbm,
                  ridx0, ridx1, cidx0, cidx1, gbuf0, gbuf1, zbuf, acc,
                  gsem0, gsem1, ssem0, ssem1, rsem0, rsem1, csem0, csem1):
    c = lax.axis_index("c")
    s = lax.axis_index("s")
    w = c * NS + s
    base = pl.multiple_of(s * S0, 8)

    z = jnp.zeros((16,), jnp.float32)

    def _zrow(j, carry):
        for i in range(D // 16):
            zbuf[j, pl.ds(i * 16, 16)] = z
        return carry

    lax.fori_loop(0, ZR, _zrow, 0)
    for t in range(S0 // ZR):
        pltpu.sync_copy(zbuf, acc.at[pl.ds(base + t * ZR, ZR)])

    @pl.when(s == 0)
    def _zero_tail():
        pltpu.sync_copy(zbuf.at[pl.ds(0, TAIL)], acc.at[pl.ds(NS * S0, TAIL)])

    plsc.subcore_barrier()

    pltpu.async_copy(row_hbm.at[w, 0], ridx0, rsem0)
    pltpu.async_copy(col_hbm.at[w, 0], cidx0, csem0)
    pltpu.async_copy(row_hbm.at[w, 1], ridx1, rsem1)
    pltpu.async_copy(col_hbm.at[w, 1], cidx1, csem1)
    pltpu.make_async_copy(row_hbm.at[w, 0], ridx0, rsem0).wait()
    pltpu.async_copy(m_hbm.at[ridx0.at[0]], gbuf0, gsem0)
    pltpu.make_async_copy(row_hbm.at[w, 1], ridx1, rsem1).wait()
    pltpu.async_copy(m_hbm.at[ridx1.at[0]], gbuf1, gsem1)

    def _pair(p, carry):
        j0 = 2 * p
        more = j0 + 2 < CH
        pltpu.make_async_copy(m_hbm.at[ridx0.at[0]], gbuf0, gsem0).wait()
        pltpu.make_async_copy(col_hbm.at[w, 0], cidx0, csem0).wait()
        pltpu.async_copy(gbuf0, acc.at[cidx0.at[0]], ssem0, add=True)

        @pl.when(more)
        def _r0():
            pltpu.async_copy(row_hbm.at[w, j0 + 2], ridx0, rsem0)

        pltpu.make_async_copy(m_hbm.at[ridx1.at[0]], gbuf1, gsem1).wait()
        pltpu.make_async_copy(col_hbm.at[w, 1], cidx1, csem1).wait()
        pltpu.async_copy(gbuf1, acc.at[cidx1.at[0]], ssem1, add=True)

        @pl.when(more)
        def _r1():
            pltpu.async_copy(row_hbm.at[w, j0 + 3], ridx1, rsem1)

        @pl.when(more)
        def _g0():
            pltpu.make_async_copy(gbuf0, acc.at[cidx0.at[0]], ssem0).wait()
            pltpu.make_async_copy(row_hbm.at[w, 0], ridx0, rsem0).wait()
            pltpu.async_copy(m_hbm.at[ridx0.at[0]], gbuf0, gsem0)
            pltpu.async_copy(col_hbm.at[w, j0 + 2], cidx0, csem0)
            pltpu.make_async_copy(gbuf1, acc.at[cidx1.at[0]], ssem1).wait()
            pltpu.make_async_copy(row_hbm.at[w, 0], ridx1, rsem1).wait()
            pltpu.async_copy(m_hbm.at[ridx1.at[0]], gbuf1, gsem1)
            pltpu.async_copy(col_hbm.at[w, j0 + 3], cidx1, csem1)

        return carry

    lax.fori_loop(0, CH // 2, _pair, 0)

    pltpu.make_async_copy(gbuf0, acc.at[cidx0.at[0]], ssem0).wait()
    pltpu.make_async_copy(gbuf1, acc.at[cidx1.at[0]], ssem1).wait()

    @pl.when(w < 4)
    def _tail_chunk():
        toff = pl.multiple_of(w * KC, 8)
        pltpu.sync_copy(trow_hbm.at[w], ridx0)
        pltpu.sync_copy(tcol_hbm.at[w], cidx0)
        pltpu.async_copy(m_hbm.at[ridx0.at[0]], gbuf0, gsem0).wait()
        pltpu.sync_copy(gbuf0, acc.at[cidx0.at[0]], add=True)

    plsc.subcore_barrier()

    off = pl.multiple_of(c * N + s * S0, 8)
    pltpu.sync_copy(acc.at[pl.ds(base, S0)], out_hbm.at[pl.ds(off, S0)])

    @pl.when(s == 0)
    def _write_tail():
        toff = pl.multiple_of(c * N + NS * S0, 8)
        pltpu.sync_copy(acc.at[pl.ds(NS * S0, TAIL)],
                        out_hbm.at[pl.ds(toff, TAIL)])


@functools.partial(
    pl.kernel,
    out_type=jax.ShapeDtypeStruct((NC * N, D), jnp.float32),
    mesh=_mesh,
    scratch_types=[
        pltpu.VMEM((NCHUNK, K), jnp.int32),
        pltpu.VMEM((K, D), jnp.float32),
        pltpu.VMEM((ZR, D), jnp.float32),
        pltpu.VMEM_SHARED((N, D), jnp.float32),
    ],
)
def _degree(col_hbm, out_hbm, cidx, ones_v, zbuf, acc):
    c = lax.axis_index("c")
    s = lax.axis_index("s")
    w = c * NS + s
    base = pl.multiple_of(s * S0, 8)

    z = jnp.zeros((16,), jnp.float32)
    o = jnp.ones((16,), jnp.float32)

    def _zrow(j, carry):
        for i in range(D // 16):
            zbuf[j, pl.ds(i * 16, 16)] = z
        return carry

    lax.fori_loop(0, ZR, _zrow, 0)

    def _orow(j, carry):
        for i in range(D // 16):
            ones_v[j, pl.ds(i * 16, 16)] = o
        return carry

    lax.fori_loop(0, K, _orow, 0)
    for t in range(S0 // ZR):
        pltpu.sync_copy(zbuf, acc.at[pl.ds(base + t * ZR, ZR)])

    @pl.when(s == 0)
    def _zero_tail():
        pltpu.sync_copy(zbuf.at[pl.ds(0, TAIL)], acc.at[pl.ds(NS * S0, TAIL)])

    plsc.subcore_barrier()

    pltpu.sync_copy(col_hbm.at[w], cidx)

    def _chunk(j, carry):
        pltpu.sync_copy(ones_v, acc.at[cidx.at[j]], add=True)
        return carry

    lax.fori_loop(0, NCHUNK, _chunk, 0)
    plsc.subcore_barrier()

    off = pl.multiple_of(c * N + s * S0, 8)
    pltpu.sync_copy(acc.at[pl.ds(base, S0)], out_hbm.at[pl.ds(off, S0)])

    @pl.when(s == 0)
    def _write_tail():
        toff = pl.multiple_of(c * N + NS * S0, 8)
        pltpu.sync_copy(acc.at[pl.ds(NS * S0, TAIL)],
                        out_hbm.at[pl.ds(toff, TAIL)])


_R = 2000


def _mm_first_body(dg_ref, x_ref, w_ref, b_ref, dinv_ref, m_ref):
    deg = dg_ref[0] + dg_ref[1]
    dinv = jnp.where(deg > 0, lax.rsqrt(deg), 0.0)
    dinv_ref[...] = dinv[:, 0:DW]
    t = jnp.dot(x_ref[...], w_ref[...], preferred_element_type=jnp.float32)
    m_ref[...] = dinv[:, 0:1] * (t + b_ref[...])


def _mm_mid_body(dinv_ref, h_ref, sp_ref, w_ref, b_ref, hn_ref, m_ref):
    dinv = dinv_ref[:, 0:1]
    hn = jnp.maximum(dinv * (sp_ref[0] + sp_ref[1]) + h_ref[...], 0.0)
    hn_ref[...] = hn
    t = jnp.dot(hn, w_ref[...], preferred_element_type=jnp.float32)
    m_ref[...] = dinv * (t + b_ref[...])


def _mm_last_body(dinv_ref, h_ref, sp_ref, w_ref, b_ref, out_ref):
    dinv = dinv_ref[:, 0:1]
    hn = jnp.maximum(dinv * (sp_ref[0] + sp_ref[1]) + h_ref[...], 0.0)
    out_ref[...] = (
        jnp.dot(hn, w_ref[...], preferred_element_type=jnp.float32)
        + b_ref[...])


def _mm_first(degp, x, w, b):
    return pl.pallas_call(
        _mm_first_body,
        grid=(N // _R,),
        in_specs=[
            pl.BlockSpec((2, _R, D), lambda i: (0, i, 0)),
            pl.BlockSpec((_R, D), lambda i: (i, 0)),
            pl.BlockSpec((D, D), lambda i: (0, 0)),
            pl.BlockSpec((1, D), lambda i: (0, 0)),
        ],
        out_specs=[
            pl.BlockSpec((_R, DW), lambda i: (i, 0)),
            pl.BlockSpec((_R, D), lambda i: (i, 0)),
        ],
        out_shape=[
            jax.ShapeDtypeStruct((N, DW), jnp.float32),
            jax.ShapeDtypeStruct((N, D), jnp.float32),
        ],
    )(degp, x, w, b)


def _mm_mid(dinv, h, sp, w, b):
    return pl.pallas_call(
        _mm_mid_body,
        grid=(N // _R,),
        in_specs=[
            pl.BlockSpec((_R, DW), lambda i: (i, 0)),
            pl.BlockSpec((_R, D), lambda i: (i, 0)),
            pl.BlockSpec((2, _R, D), lambda i: (0, i, 0)),
            pl.BlockSpec((D, D), lambda i: (0, 0)),
            pl.BlockSpec((1, D), lambda i: (0, 0)),
        ],
        out_specs=[
            pl.BlockSpec((_R, D), lambda i: (i, 0)),
            pl.BlockSpec((_R, D), lambda i: (i, 0)),
        ],
        out_shape=[
            jax.ShapeDtypeStruct((N, D), jnp.float32),
            jax.ShapeDtypeStruct((N, D), jnp.float32),
        ],
    )(dinv, h, sp, w, b)


def _mm_last(dinv, h, sp, w, b):
    return pl.pallas_call(
        _mm_last_body,
        grid=(N // _R,),
        in_specs=[
            pl.BlockSpec((_R, DW), lambda i: (i, 0)),
            pl.BlockSpec((_R, D), lambda i: (i, 0)),
            pl.BlockSpec((2, _R, D), lambda i: (0, i, 0)),
            pl.BlockSpec((D, D), lambda i: (0, 0)),
            pl.BlockSpec((1, D), lambda i: (0, 0)),
        ],
        out_specs=pl.BlockSpec((_R, D), lambda i: (i, 0)),
        out_shape=jax.ShapeDtypeStruct((N, D), jnp.float32),
    )(dinv, h, sp, w, b)


def kernel(x, edge_index, W1, b1, W2, b2, W3, b3, Wh, bh):
    row_fl = edge_index[0]
    col_fl = edge_index[1]
    nmain = NW * CH * KC
    row2 = row_fl[:nmain].reshape(NW, CH, 1, KC)
    col2 = col_fl[:nmain].reshape(NW, CH, 1, KC)
    trow = row_fl[nmain:].reshape(NTAIL, 1, KC)
    tcol = col_fl[nmain:].reshape(NTAIL, 1, KC)
    col = col_fl.reshape(NW, NCHUNK, K)
    b1r = b1.reshape(1, D)
    b2r = b2.reshape(1, D)
    b3r = b3.reshape(1, D)
    bhr = bh.reshape(1, D)

    degp = _degree(col).reshape(NC, N, D)
    dinv, m = _mm_first(degp, x, W1, b1r)
    h = x
    for w, b in ((W2, b2r), (W3, b3r)):
        sp = _scatter_rows(m, row2, col2, trow, tcol).reshape(NC, N, D)
        h, m = _mm_mid(dinv, h, sp, w, b)
    sp = _scatter_rows(m, row2, col2, trow, tcol).reshape(NC, N, D)
    return _mm_last(dinv, h, sp, Wh, bhr)


# speedup vs baseline: 18.1791x; 1.1724x over previous
import functools

import jax
import jax.numpy as jnp
from jax import lax
from jax.experimental import pallas as pl
from jax.experimental.pallas import tpu as pltpu
from jax.experimental.pallas import tpu_sc as plsc

N = 10000
E = 320000
D = 128

NC, NS = 2, 16
NW = NC * NS
K = 80
NCHUNK = E // (NW * K)
KC = 64
CH = 156
NTAIL = (E - NW * CH * KC) // KC
S0 = 624
TAIL = N - NS * S0
ZR = 16
DW = 16

_mesh = plsc.VectorSubcoreMesh(core_axis_name="c", subcore_axis_name="s")


NBUF = 4


@functools.partial(
    pl.kernel,
    out_type=jax.ShapeDtypeStruct((NC * N, D), jnp.float32),
    mesh=_mesh,
    scratch_types=(
        [pltpu.VMEM((1, KC), jnp.int32) for _ in range(NBUF)] +
        [pltpu.VMEM((1, KC), jnp.int32) for _ in range(NBUF)] +
        [pltpu.VMEM((KC, D), jnp.float32) for _ in range(NBUF)] +
        [pltpu.VMEM((ZR, D), jnp.float32),
         pltpu.VMEM_SHARED((N, D), jnp.float32)] +
        [pltpu.SemaphoreType.DMA for _ in range(4 * NBUF)]
    ),
)
def _scatter_rows(m_hbm, row_hbm, col_hbm, trow_hbm, tcol_hbm, out_hbm,
                  *bufs):
    ridx = bufs[0:NBUF]
    cidx = bufs[NBUF:2 * NBUF]
    gbuf = bufs[2 * NBUF:3 * NBUF]
    zbuf = bufs[3 * NBUF]
    acc = bufs[3 * NBUF + 1]
    gsem = bufs[3 * NBUF + 2:3 * NBUF + 2 + NBUF]
    ssem = bufs[3 * NBUF + 2 + NBUF:3 * NBUF + 2 + 2 * NBUF]
    rsem = bufs[3 * NBUF + 2 + 2 * NBUF:3 * NBUF + 2 + 3 * NBUF]
    csem = bufs[3 * NBUF + 2 + 3 * NBUF:3 * NBUF + 2 + 4 * NBUF]

    c = lax.axis_index("c")
    s = lax.axis_index("s")
    w = c * NS + s
    base = pl.multiple_of(s * S0, 8)

    z = jnp.zeros((16,), jnp.float32)

    def _zrow(j, carry):
        for i in range(D // 16):
            zbuf[j, pl.ds(i * 16, 16)] = z
        return carry

    lax.fori_loop(0, ZR, _zrow, 0)
    for t in range(S0 // ZR):
        pltpu.sync_copy(zbuf, acc.at[pl.ds(base + t * ZR, ZR)])

    @pl.when(s == 0)
    def _zero_tail():
        pltpu.sync_copy(zbuf.at[pl.ds(0, TAIL)], acc.at[pl.ds(NS * S0, TAIL)])

    plsc.subcore_barrier()

    for b in range(NBUF):
        pltpu.async_copy(row_hbm.at[w, b], ridx[b], rsem[b])
        pltpu.async_copy(col_hbm.at[w, b], cidx[b], csem[b])
    for b in range(NBUF):
        pltpu.make_async_copy(row_hbm.at[w, 0], ridx[b], rsem[b]).wait()
        pltpu.async_copy(m_hbm.at[ridx[b].at[0]], gbuf[b], gsem[b])

    def _group(g, carry):
        j0 = g * NBUF
        for b in range(NBUF):
            pltpu.make_async_copy(m_hbm.at[ridx[b].at[0]], gbuf[b],
                                  gsem[b]).wait()
            pltpu.make_async_copy(col_hbm.at[w, 0], cidx[b], csem[b]).wait()
            pltpu.async_copy(gbuf[b], acc.at[cidx[b].at[0]], ssem[b],
                             add=True)

            @pl.when(j0 + b + NBUF < CH)
            def _r():
                pltpu.async_copy(row_hbm.at[w, j0 + b + NBUF], ridx[b],
                                 rsem[b])

        for b in range(NBUF):
            @pl.when(j0 + b + NBUF < CH)
            def _g():
                pltpu.make_async_copy(gbuf[b], acc.at[cidx[b].at[0]],
                                      ssem[b]).wait()
                pltpu.make_async_copy(row_hbm.at[w, 0], ridx[b],
                                      rsem[b]).wait()
                pltpu.async_copy(m_hbm.at[ridx[b].at[0]], gbuf[b], gsem[b])
                pltpu.async_copy(col_hbm.at[w, j0 + b + NBUF], cidx[b],
                                 csem[b])

        return carry

    lax.fori_loop(0, CH // NBUF, _group, 0)

    for b in range(NBUF):
        pltpu.make_async_copy(gbuf[b], acc.at[cidx[b].at[0]], ssem[b]).wait()

    @pl.when(w < NTAIL)
    def _tail_chunk():
        pltpu.sync_copy(trow_hbm.at[w], ridx[0])
        pltpu.sync_copy(tcol_hbm.at[w], cidx[0])
        pltpu.async_copy(m_hbm.at[ridx[0].at[0]], gbuf[0], gsem[0]).wait()
        pltpu.sync_copy(gbuf[0], acc.at[cidx[0].at[0]], add=True)

    plsc.subcore_barrier()

    off = pl.multiple_of(c * N + s * S0, 8)
    pltpu.sync_copy(acc.at[pl.ds(base, S0)], out_hbm.at[pl.ds(off, S0)])

    @pl.when(s == 0)
    def _write_tail():
        toff = pl.multiple_of(c * N + NS * S0, 8)
        pltpu.sync_copy(acc.at[pl.ds(NS * S0, TAIL)],
                        out_hbm.at[pl.ds(toff, TAIL)])


@functools.partial(
    pl.kernel,
    out_type=jax.ShapeDtypeStruct((NC * N, D), jnp.float32),
    mesh=_mesh,
    scratch_types=[
        pltpu.VMEM((NCHUNK, K), jnp.int32),
        pltpu.VMEM((K, D), jnp.float32),
        pltpu.VMEM((ZR, D), jnp.float32),
        pltpu.VMEM_SHARED((N, D), jnp.float32),
    ],
)
def _degree(col_hbm, out_hbm, cidx, ones_v, zbuf, acc):
    c = lax.axis_index("c")
    s = lax.axis_index("s")
    w = c * NS + s
    base = pl.multiple_of(s * S0, 8)

    z = jnp.zeros((16,), jnp.float32)
    o = jnp.ones((16,), jnp.float32)

    def _zrow(j, carry):
        for i in range(D // 16):
            zbuf[j, pl.ds(i * 16, 16)] = z
        return carry

    lax.fori_loop(0, ZR, _zrow, 0)

    def _orow(j, carry):
        for i in range(D // 16):
            ones_v[j, pl.ds(i * 16, 16)] = o
        return carry

    lax.fori_loop(0, K, _orow, 0)
    for t in range(S0 // ZR):
        pltpu.sync_copy(zbuf, acc.at[pl.ds(base + t * ZR, ZR)])

    @pl.when(s == 0)
    def _zero_tail():
        pltpu.sync_copy(zbuf.at[pl.ds(0, TAIL)], acc.at[pl.ds(NS * S0, TAIL)])

    plsc.subcore_barrier()

    pltpu.sync_copy(col_hbm.at[w], cidx)

    def _chunk(j, carry):
        pltpu.sync_copy(ones_v, acc.at[cidx.at[j]], add=True)
        return carry

    lax.fori_loop(0, NCHUNK, _chunk, 0)
    plsc.subcore_barrier()

    off = pl.multiple_of(c * N + s * S0, 8)
    pltpu.sync_copy(acc.at[pl.ds(base, S0)], out_hbm.at[pl.ds(off, S0)])

    @pl.when(s == 0)
    def _write_tail():
        toff = pl.multiple_of(c * N + NS * S0, 8)
        pltpu.sync_copy(acc.at[pl.ds(NS * S0, TAIL)],
                        out_hbm.at[pl.ds(toff, TAIL)])


_R = 2000


def _mm_first_body(dg_ref, x_ref, w_ref, b_ref, dinv_ref, m_ref):
    deg = dg_ref[0] + dg_ref[1]
    dinv = jnp.where(deg > 0, lax.rsqrt(deg), 0.0)
    dinv_ref[...] = dinv[:, 0:DW]
    t = jnp.dot(x_ref[...], w_ref[...], preferred_element_type=jnp.float32)
    m_ref[...] = dinv[:, 0:1] * (t + b_ref[...])


def _mm_mid_body(dinv_ref, h_ref, sp_ref, w_ref, b_ref, hn_ref, m_ref):
    dinv = dinv_ref[:, 0:1]
    hn = jnp.maximum(dinv * (sp_ref[0] + sp_ref[1]) + h_ref[...], 0.0)
    hn_ref[...] = hn
    t = jnp.dot(hn, w_ref[...], preferred_element_type=jnp.float32)
    m_ref[...] = dinv * (t + b_ref[...])


def _mm_last_body(dinv_ref, h_ref, sp_ref, w_ref, b_ref, out_ref):
    dinv = dinv_ref[:, 0:1]
    hn = jnp.maximum(dinv * (sp_ref[0] + sp_ref[1]) + h_ref[...], 0.0)
    out_ref[...] = (
        jnp.dot(hn, w_ref[...], preferred_element_type=jnp.float32)
        + b_ref[...])


def _mm_first(degp, x, w, b):
    return pl.pallas_call(
        _mm_first_body,
        grid=(N // _R,),
        in_specs=[
            pl.BlockSpec((2, _R, D), lambda i: (0, i, 0)),
            pl.BlockSpec((_R, D), lambda i: (i, 0)),
            pl.BlockSpec((D, D), lambda i: (0, 0)),
            pl.BlockSpec((1, D), lambda i: (0, 0)),
        ],
        out_specs=[
            pl.BlockSpec((_R, DW), lambda i: (i, 0)),
            pl.BlockSpec((_R, D), lambda i: (i, 0)),
        ],
        out_shape=[
            jax.ShapeDtypeStruct((N, DW), jnp.float32),
            jax.ShapeDtypeStruct((N, D), jnp.float32),
        ],
    )(degp, x, w, b)


def _mm_mid(dinv, h, sp, w, b):
    return pl.pallas_call(
        _mm_mid_body,
        grid=(N // _R,),
        in_specs=[
            pl.BlockSpec((_R, DW), lambda i: (i, 0)),
            pl.BlockSpec((_R, D), lambda i: (i, 0)),
            pl.BlockSpec((2, _R, D), lambda i: (0, i, 0)),
            pl.BlockSpec((D, D), lambda i: (0, 0)),
            pl.BlockSpec((1, D), lambda i: (0, 0)),
        ],
        out_specs=[
            pl.BlockSpec((_R, D), lambda i: (i, 0)),
            pl.BlockSpec((_R, D), lambda i: (i, 0)),
        ],
        out_shape=[
            jax.ShapeDtypeStruct((N, D), jnp.float32),
            jax.ShapeDtypeStruct((N, D), jnp.float32),
        ],
    )(dinv, h, sp, w, b)


def _mm_last(dinv, h, sp, w, b):
    return pl.pallas_call(
        _mm_last_body,
        grid=(N // _R,),
        in_specs=[
            pl.BlockSpec((_R, DW), lambda i: (i, 0)),
            pl.BlockSpec((_R, D), lambda i: (i, 0)),
            pl.BlockSpec((2, _R, D), lambda i: (0, i, 0)),
            pl.BlockSpec((D, D), lambda i: (0, 0)),
            pl.BlockSpec((1, D), lambda i: (0, 0)),
        ],
        out_specs=pl.BlockSpec((_R, D), lambda i: (i, 0)),
        out_shape=jax.ShapeDtypeStruct((N, D), jnp.float32),
    )(dinv, h, sp, w, b)


def kernel(x, edge_index, W1, b1, W2, b2, W3, b3, Wh, bh):
    row_fl = edge_index[0]
    col_fl = edge_index[1]
    nmain = NW * CH * KC
    row2 = row_fl[:nmain].reshape(NW, CH, 1, KC)
    col2 = col_fl[:nmain].reshape(NW, CH, 1, KC)
    trow = row_fl[nmain:].reshape(NTAIL, 1, KC)
    tcol = col_fl[nmain:].reshape(NTAIL, 1, KC)
    col = col_fl.reshape(NW, NCHUNK, K)
    b1r = b1.reshape(1, D)
    b2r = b2.reshape(1, D)
    b3r = b3.reshape(1, D)
    bhr = bh.reshape(1, D)

    degp = _degree(col).reshape(NC, N, D)
    dinv, m = _mm_first(degp, x, W1, b1r)
    h = x
    for w, b in ((W2, b2r), (W3, b3r)):
        sp = _scatter_rows(m, row2, col2, trow, tcol).reshape(NC, N, D)
        h, m = _mm_mid(dinv, h, sp, w, b)
    sp = _scatter_rows(m, row2, col2, trow, tcol).reshape(NC, N, D)
    return _mm_last(dinv, h, sp, Wh, bhr)


# speedup vs baseline: 18.4471x; 1.0147x over previous
import functools

import jax
import jax.numpy as jnp
from jax import lax
from jax.experimental import pallas as pl
from jax.experimental.pallas import tpu as pltpu
from jax.experimental.pallas import tpu_sc as plsc

N = 10000
E = 320000
D = 128

NC, NS = 2, 16
NW = NC * NS
K = 80
NCHUNK = E // (NW * K)
KC = 64
CH = 156
NTAIL = (E - NW * CH * KC) // KC
S0 = 624
TAIL = N - NS * S0
ZR = 104
DW = 16

_mesh = plsc.VectorSubcoreMesh(core_axis_name="c", subcore_axis_name="s")


NBUF = 4


@functools.partial(
    pl.kernel,
    out_type=jax.ShapeDtypeStruct((NC * N, D), jnp.float32),
    mesh=_mesh,
    scratch_types=(
        [pltpu.VMEM((1, KC), jnp.int32) for _ in range(NBUF)] +
        [pltpu.VMEM((1, KC), jnp.int32) for _ in range(NBUF)] +
        [pltpu.VMEM((KC, D), jnp.float32) for _ in range(NBUF)] +
        [pltpu.VMEM((ZR, D), jnp.float32),
         pltpu.VMEM_SHARED((N, D), jnp.float32)] +
        [pltpu.SemaphoreType.DMA for _ in range(4 * NBUF)]
    ),
)
def _scatter_rows(m_hbm, row_hbm, col_hbm, trow_hbm, tcol_hbm, out_hbm,
                  *bufs):
    ridx = bufs[0:NBUF]
    cidx = bufs[NBUF:2 * NBUF]
    gbuf = bufs[2 * NBUF:3 * NBUF]
    zbuf = bufs[3 * NBUF]
    acc = bufs[3 * NBUF + 1]
    gsem = bufs[3 * NBUF + 2:3 * NBUF + 2 + NBUF]
    ssem = bufs[3 * NBUF + 2 + NBUF:3 * NBUF + 2 + 2 * NBUF]
    rsem = bufs[3 * NBUF + 2 + 2 * NBUF:3 * NBUF + 2 + 3 * NBUF]
    csem = bufs[3 * NBUF + 2 + 3 * NBUF:3 * NBUF + 2 + 4 * NBUF]

    c = lax.axis_index("c")
    s = lax.axis_index("s")
    w = c * NS + s
    base = pl.multiple_of(s * S0, 8)

    z = jnp.zeros((16,), jnp.float32)

    def _zrow(j, carry):
        for i in range(D // 16):
            zbuf[j, pl.ds(i * 16, 16)] = z
        return carry

    lax.fori_loop(0, ZR, _zrow, 0)
    for t in range(S0 // ZR):
        pltpu.sync_copy(zbuf, acc.at[pl.ds(base + t * ZR, ZR)])

    @pl.when(s == 0)
    def _zero_tail():
        pltpu.sync_copy(zbuf.at[pl.ds(0, TAIL)], acc.at[pl.ds(NS * S0, TAIL)])

    plsc.subcore_barrier()

    for b in range(NBUF):
        pltpu.async_copy(row_hbm.at[w, b], ridx[b], rsem[b])
        pltpu.async_copy(col_hbm.at[w, b], cidx[b], csem[b])
    for b in range(NBUF):
        pltpu.make_async_copy(row_hbm.at[w, 0], ridx[b], rsem[b]).wait()
        pltpu.async_copy(m_hbm.at[ridx[b].at[0]], gbuf[b], gsem[b])

    def _group(g, carry):
        j0 = g * NBUF
        for b in range(NBUF):
            pltpu.make_async_copy(m_hbm.at[ridx[b].at[0]], gbuf[b],
                                  gsem[b]).wait()
            pltpu.make_async_copy(col_hbm.at[w, 0], cidx[b], csem[b]).wait()
            pltpu.async_copy(gbuf[b], acc.at[cidx[b].at[0]], ssem[b],
                             add=True)

            @pl.when(j0 + b + NBUF < CH)
            def _r():
                pltpu.async_copy(row_hbm.at[w, j0 + b + NBUF], ridx[b],
                                 rsem[b])

        for b in range(NBUF):
            @pl.when(j0 + b + NBUF < CH)
            def _g():
                pltpu.make_async_copy(gbuf[b], acc.at[cidx[b].at[0]],
                                      ssem[b]).wait()
                pltpu.make_async_copy(row_hbm.at[w, 0], ridx[b],
                                      rsem[b]).wait()
                pltpu.async_copy(m_hbm.at[ridx[b].at[0]], gbuf[b], gsem[b])
                pltpu.async_copy(col_hbm.at[w, j0 + b + NBUF], cidx[b],
                                 csem[b])

        return carry

    lax.fori_loop(0, CH // NBUF, _group, 0)

    for b in range(NBUF):
        pltpu.make_async_copy(gbuf[b], acc.at[cidx[b].at[0]], ssem[b]).wait()

    @pl.when(w < NTAIL)
    def _tail_chunk():
        pltpu.sync_copy(trow_hbm.at[w], ridx[0])
        pltpu.sync_copy(tcol_hbm.at[w], cidx[0])
        pltpu.async_copy(m_hbm.at[ridx[0].at[0]], gbuf[0], gsem[0]).wait()
        pltpu.sync_copy(gbuf[0], acc.at[cidx[0].at[0]], add=True)

    plsc.subcore_barrier()

    off = pl.multiple_of(c * N + s * S0, 8)
    pltpu.sync_copy(acc.at[pl.ds(base, S0)], out_hbm.at[pl.ds(off, S0)])

    @pl.when(s == 0)
    def _write_tail():
        toff = pl.multiple_of(c * N + NS * S0, 8)
        pltpu.sync_copy(acc.at[pl.ds(NS * S0, TAIL)],
                        out_hbm.at[pl.ds(toff, TAIL)])


@functools.partial(
    pl.kernel,
    out_type=jax.ShapeDtypeStruct((NC * N, D), jnp.float32),
    mesh=_mesh,
    scratch_types=[
        pltpu.VMEM((NCHUNK, K), jnp.int32),
        pltpu.VMEM((K, D), jnp.float32),
        pltpu.VMEM((ZR, D), jnp.float32),
        pltpu.VMEM_SHARED((N, D), jnp.float32),
    ],
)
def _degree(col_hbm, out_hbm, cidx, ones_v, zbuf, acc):
    c = lax.axis_index("c")
    s = lax.axis_index("s")
    w = c * NS + s
    base = pl.multiple_of(s * S0, 8)

    z = jnp.zeros((16,), jnp.float32)
    o = jnp.ones((16,), jnp.float32)

    def _zrow(j, carry):
        for i in range(D // 16):
            zbuf[j, pl.ds(i * 16, 16)] = z
        return carry

    lax.fori_loop(0, ZR, _zrow, 0)

    def _orow(j, carry):
        for i in range(D // 16):
            ones_v[j, pl.ds(i * 16, 16)] = o
        return carry

    lax.fori_loop(0, K, _orow, 0)
    for t in range(S0 // ZR):
        pltpu.sync_copy(zbuf, acc.at[pl.ds(base + t * ZR, ZR)])

    @pl.when(s == 0)
    def _zero_tail():
        pltpu.sync_copy(zbuf.at[pl.ds(0, TAIL)], acc.at[pl.ds(NS * S0, TAIL)])

    plsc.subcore_barrier()

    pltpu.sync_copy(col_hbm.at[w], cidx)

    def _chunk(j, carry):
        pltpu.sync_copy(ones_v, acc.at[cidx.at[j]], add=True)
        return carry

    lax.fori_loop(0, NCHUNK, _chunk, 0)
    plsc.subcore_barrier()

    off = pl.multiple_of(c * N + s * S0, 8)
    pltpu.sync_copy(acc.at[pl.ds(base, S0)], out_hbm.at[pl.ds(off, S0)])

    @pl.when(s == 0)
    def _write_tail():
        toff = pl.multiple_of(c * N + NS * S0, 8)
        pltpu.sync_copy(acc.at[pl.ds(NS * S0, TAIL)],
                        out_hbm.at[pl.ds(toff, TAIL)])


_R = 2000


def _mm_plain_body(x_ref, w_ref, b_ref, t_ref):
    t_ref[...] = (
        jnp.dot(x_ref[...], w_ref[...], preferred_element_type=jnp.float32)
        + b_ref[...])


def _mm_scale_body(dg_ref, t_ref, dinv_ref, m_ref):
    deg = dg_ref[0] + dg_ref[1]
    dinv = jnp.where(deg > 0, lax.rsqrt(deg), 0.0)
    dinv_ref[...] = dinv[:, 0:DW]
    m_ref[...] = dinv[:, 0:1] * t_ref[...]


def _mm_mid_body(dinv_ref, h_ref, sp_ref, w_ref, b_ref, hn_ref, m_ref):
    dinv = dinv_ref[:, 0:1]
    hn = jnp.maximum(dinv * (sp_ref[0] + sp_ref[1]) + h_ref[...], 0.0)
    hn_ref[...] = hn
    t = jnp.dot(hn, w_ref[...], preferred_element_type=jnp.float32)
    m_ref[...] = dinv * (t + b_ref[...])


def _mm_last_body(dinv_ref, h_ref, sp_ref, w_ref, b_ref, out_ref):
    dinv = dinv_ref[:, 0:1]
    hn = jnp.maximum(dinv * (sp_ref[0] + sp_ref[1]) + h_ref[...], 0.0)
    out_ref[...] = (
        jnp.dot(hn, w_ref[...], preferred_element_type=jnp.float32)
        + b_ref[...])


def _mm_plain(x, w, b):
    return pl.pallas_call(
        _mm_plain_body,
        grid=(N // _R,),
        in_specs=[
            pl.BlockSpec((_R, D), lambda i: (i, 0)),
            pl.BlockSpec((D, D), lambda i: (0, 0)),
            pl.BlockSpec((1, D), lambda i: (0, 0)),
        ],
        out_specs=pl.BlockSpec((_R, D), lambda i: (i, 0)),
        out_shape=jax.ShapeDtypeStruct((N, D), jnp.float32),
    )(x, w, b)


def _mm_scale(degp, t):
    return pl.pallas_call(
        _mm_scale_body,
        grid=(N // _R,),
        in_specs=[
            pl.BlockSpec((2, _R, D), lambda i: (0, i, 0)),
            pl.BlockSpec((_R, D), lambda i: (i, 0)),
        ],
        out_specs=[
            pl.BlockSpec((_R, DW), lambda i: (i, 0)),
            pl.BlockSpec((_R, D), lambda i: (i, 0)),
        ],
        out_shape=[
            jax.ShapeDtypeStruct((N, DW), jnp.float32),
            jax.ShapeDtypeStruct((N, D), jnp.float32),
        ],
    )(degp, t)


def _mm_mid(dinv, h, sp, w, b):
    return pl.pallas_call(
        _mm_mid_body,
        grid=(N // _R,),
        in_specs=[
            pl.BlockSpec((_R, DW), lambda i: (i, 0)),
            pl.BlockSpec((_R, D), lambda i: (i, 0)),
            pl.BlockSpec((2, _R, D), lambda i: (0, i, 0)),
            pl.BlockSpec((D, D), lambda i: (0, 0)),
            pl.BlockSpec((1, D), lambda i: (0, 0)),
        ],
        out_specs=[
            pl.BlockSpec((_R, D), lambda i: (i, 0)),
            pl.BlockSpec((_R, D), lambda i: (i, 0)),
        ],
        out_shape=[
            jax.ShapeDtypeStruct((N, D), jnp.float32),
            jax.ShapeDtypeStruct((N, D), jnp.float32),
        ],
    )(dinv, h, sp, w, b)


def _mm_last(dinv, h, sp, w, b):
    return pl.pallas_call(
        _mm_last_body,
        grid=(N // _R,),
        in_specs=[
            pl.BlockSpec((_R, DW), lambda i: (i, 0)),
            pl.BlockSpec((_R, D), lambda i: (i, 0)),
            pl.BlockSpec((2, _R, D), lambda i: (0, i, 0)),
            pl.BlockSpec((D, D), lambda i: (0, 0)),
            pl.BlockSpec((1, D), lambda i: (0, 0)),
        ],
        out_specs=pl.BlockSpec((_R, D), lambda i: (i, 0)),
        out_shape=jax.ShapeDtypeStruct((N, D), jnp.float32),
    )(dinv, h, sp, w, b)


def kernel(x, edge_index, W1, b1, W2, b2, W3, b3, Wh, bh):
    row_fl = edge_index[0]
    col_fl = edge_index[1]
    nmain = NW * CH * KC
    row2 = row_fl[:nmain].reshape(NW, CH, 1, KC)
    col2 = col_fl[:nmain].reshape(NW, CH, 1, KC)
    trow = row_fl[nmain:].reshape(NTAIL, 1, KC)
    tcol = col_fl[nmain:].reshape(NTAIL, 1, KC)
    col = col_fl.reshape(NW, NCHUNK, K)
    b1r = b1.reshape(1, D)
    b2r = b2.reshape(1, D)
    b3r = b3.reshape(1, D)
    bhr = bh.reshape(1, D)

    t1 = _mm_plain(x, W1, b1r)
    degp = _degree(col).reshape(NC, N, D)
    dinv, m = _mm_scale(degp, t1)
    h = x
    for w, b in ((W2, b2r), (W3, b3r)):
        sp = _scatter_rows(m, row2, col2, trow, tcol).reshape(NC, N, D)
        h, m = _mm_mid(dinv, h, sp, w, b)
    sp = _scatter_rows(m, row2, col2, trow, tcol).reshape(NC, N, D)
    return _mm_last(dinv, h, sp, Wh, bhr)


# speedup vs baseline: 19.4371x; 1.0537x over previous
import functools

import jax
import jax.numpy as jnp
from jax import lax
from jax.experimental import pallas as pl
from jax.experimental.pallas import tpu as pltpu
from jax.experimental.pallas import tpu_sc as plsc

N = 10000
E = 320000
D = 128

NC, NS = 2, 16
NW = NC * NS
K = 80
NCHUNK = E // (NW * K)
KC = 64
CH = 156
NTAIL = (E - NW * CH * KC) // KC
S0 = 624
TAIL = N - NS * S0
ZR = 104
DW = 16

_mesh = plsc.VectorSubcoreMesh(core_axis_name="c", subcore_axis_name="s")


NBUF = 4


@functools.partial(
    pl.kernel,
    out_type=jax.ShapeDtypeStruct((NC * N, D), jnp.float32),
    mesh=_mesh,
    scratch_types=(
        [pltpu.VMEM((1, KC), jnp.int32) for _ in range(NBUF)] +
        [pltpu.VMEM((1, KC), jnp.int32) for _ in range(NBUF)] +
        [pltpu.VMEM((KC, D), jnp.float32) for _ in range(NBUF)] +
        [pltpu.VMEM((ZR, D), jnp.float32),
         pltpu.VMEM_SHARED((N, D), jnp.float32)] +
        [pltpu.SemaphoreType.DMA for _ in range(4 * NBUF)]
    ),
)
def _scatter_rows(m_hbm, row_hbm, col_hbm, trow_hbm, tcol_hbm, out_hbm,
                  *bufs):
    ridx = bufs[0:NBUF]
    cidx = bufs[NBUF:2 * NBUF]
    gbuf = bufs[2 * NBUF:3 * NBUF]
    zbuf = bufs[3 * NBUF]
    acc = bufs[3 * NBUF + 1]
    gsem = bufs[3 * NBUF + 2:3 * NBUF + 2 + NBUF]
    ssem = bufs[3 * NBUF + 2 + NBUF:3 * NBUF + 2 + 2 * NBUF]
    rsem = bufs[3 * NBUF + 2 + 2 * NBUF:3 * NBUF + 2 + 3 * NBUF]
    csem = bufs[3 * NBUF + 2 + 3 * NBUF:3 * NBUF + 2 + 4 * NBUF]

    c = lax.axis_index("c")
    s = lax.axis_index("s")
    w = c * NS + s
    base = pl.multiple_of(s * S0, 8)

    z = jnp.zeros((16,), jnp.float32)

    def _zrow(j, carry):
        for i in range(D // 16):
            zbuf[j, pl.ds(i * 16, 16)] = z
        return carry

    lax.fori_loop(0, ZR, _zrow, 0)
    for t in range(S0 // ZR):
        pltpu.sync_copy(zbuf, acc.at[pl.ds(base + t * ZR, ZR)])

    @pl.when(s == 0)
    def _zero_tail():
        pltpu.sync_copy(zbuf.at[pl.ds(0, TAIL)], acc.at[pl.ds(NS * S0, TAIL)])

    plsc.subcore_barrier()

    for b in range(NBUF):
        pltpu.async_copy(row_hbm.at[w, b], ridx[b], rsem[b])
        pltpu.async_copy(col_hbm.at[w, b], cidx[b], csem[b])
    for b in range(NBUF):
        pltpu.make_async_copy(row_hbm.at[w, 0], ridx[b], rsem[b]).wait()
        pltpu.async_copy(m_hbm.at[ridx[b].at[0]], gbuf[b], gsem[b])

    def _group(g, carry):
        j0 = g * NBUF
        for b in range(NBUF):
            pltpu.make_async_copy(m_hbm.at[ridx[b].at[0]], gbuf[b],
                                  gsem[b]).wait()
            pltpu.make_async_copy(col_hbm.at[w, 0], cidx[b], csem[b]).wait()
            pltpu.async_copy(gbuf[b], acc.at[cidx[b].at[0]], ssem[b],
                             add=True)

            @pl.when(j0 + b + NBUF < CH)
            def _r():
                pltpu.async_copy(row_hbm.at[w, j0 + b + NBUF], ridx[b],
                                 rsem[b])

        for b in range(NBUF):
            @pl.when(j0 + b + NBUF < CH)
            def _g():
                pltpu.make_async_copy(gbuf[b], acc.at[cidx[b].at[0]],
                                      ssem[b]).wait()
                pltpu.make_async_copy(row_hbm.at[w, 0], ridx[b],
                                      rsem[b]).wait()
                pltpu.async_copy(m_hbm.at[ridx[b].at[0]], gbuf[b], gsem[b])
                pltpu.async_copy(col_hbm.at[w, j0 + b + NBUF], cidx[b],
                                 csem[b])

        return carry

    lax.fori_loop(0, CH // NBUF, _group, 0)

    for b in range(NBUF):
        pltpu.make_async_copy(gbuf[b], acc.at[cidx[b].at[0]], ssem[b]).wait()

    @pl.when(w < NTAIL)
    def _tail_chunk():
        pltpu.sync_copy(trow_hbm.at[w], ridx[0])
        pltpu.sync_copy(tcol_hbm.at[w], cidx[0])
        pltpu.async_copy(m_hbm.at[ridx[0].at[0]], gbuf[0], gsem[0]).wait()
        pltpu.sync_copy(gbuf[0], acc.at[cidx[0].at[0]], add=True)

    plsc.subcore_barrier()

    off = pl.multiple_of(c * N + s * S0, 8)
    pltpu.sync_copy(acc.at[pl.ds(base, S0)], out_hbm.at[pl.ds(off, S0)])

    @pl.when(s == 0)
    def _write_tail():
        toff = pl.multiple_of(c * N + NS * S0, 8)
        pltpu.sync_copy(acc.at[pl.ds(NS * S0, TAIL)],
                        out_hbm.at[pl.ds(toff, TAIL)])


NP_ = 10240
SD = NP_ // NS


@functools.partial(
    pl.kernel,
    out_type=jax.ShapeDtypeStruct((NC * NP_,), jnp.float32),
    mesh=_mesh,
    scratch_types=[
        pltpu.VMEM((NCHUNK, K), jnp.int32),
        pltpu.VMEM((K,), jnp.float32),
        pltpu.VMEM((SD,), jnp.float32),
        pltpu.VMEM_SHARED((NP_,), jnp.float32),
    ],
)
def _degree(col_hbm, out_hbm, cidx, ones_v, zbuf, acc):
    c = lax.axis_index("c")
    s = lax.axis_index("s")
    w = c * NS + s
    base = pl.multiple_of(s * SD, 8)

    z = jnp.zeros((16,), jnp.float32)
    o = jnp.ones((16,), jnp.float32)

    def _zrow(j, carry):
        zbuf[pl.ds(j * 16, 16)] = z
        return carry

    lax.fori_loop(0, SD // 16, _zrow, 0)
    for i in range(K // 16):
        ones_v[pl.ds(i * 16, 16)] = o
    pltpu.sync_copy(zbuf, acc.at[pl.ds(base, SD)])
    plsc.subcore_barrier()

    pltpu.sync_copy(col_hbm.at[w], cidx)

    def _chunk(j, carry):
        pltpu.sync_copy(ones_v, acc.at[cidx.at[j]], add=True)
        return carry

    lax.fori_loop(0, NCHUNK, _chunk, 0)
    plsc.subcore_barrier()

    off = pl.multiple_of(c * NP_ + s * SD, 8)
    pltpu.sync_copy(acc.at[pl.ds(base, SD)], out_hbm.at[pl.ds(off, SD)])


_R = 2000


def _mm_plain_body(x_ref, w_ref, b_ref, t_ref):
    t_ref[...] = (
        jnp.dot(x_ref[...], w_ref[...], preferred_element_type=jnp.float32)
        + b_ref[...])


def _mm_scale_body(dg_ref, t_ref, dinv_ref, m_ref):
    deg = dg_ref[0] + dg_ref[1]
    dinv = jnp.where(deg > 0, lax.rsqrt(deg), 0.0)
    dinv_ref[...] = jnp.broadcast_to(dinv, (dinv.shape[0], DW))
    m_ref[...] = dinv * t_ref[...]


def _mm_mid_body(dinv_ref, h_ref, sp_ref, w_ref, b_ref, hn_ref, m_ref):
    dinv = dinv_ref[:, 0:1]
    hn = jnp.maximum(dinv * (sp_ref[0] + sp_ref[1]) + h_ref[...], 0.0)
    hn_ref[...] = hn
    t = jnp.dot(hn, w_ref[...], preferred_element_type=jnp.float32)
    m_ref[...] = dinv * (t + b_ref[...])


def _mm_last_body(dinv_ref, h_ref, sp_ref, w_ref, b_ref, out_ref):
    dinv = dinv_ref[:, 0:1]
    hn = jnp.maximum(dinv * (sp_ref[0] + sp_ref[1]) + h_ref[...], 0.0)
    out_ref[...] = (
        jnp.dot(hn, w_ref[...], preferred_element_type=jnp.float32)
        + b_ref[...])


def _mm_plain(x, w, b):
    return pl.pallas_call(
        _mm_plain_body,
        grid=(N // _R,),
        in_specs=[
            pl.BlockSpec((_R, D), lambda i: (i, 0)),
            pl.BlockSpec((D, D), lambda i: (0, 0)),
            pl.BlockSpec((1, D), lambda i: (0, 0)),
        ],
        out_specs=pl.BlockSpec((_R, D), lambda i: (i, 0)),
        out_shape=jax.ShapeDtypeStruct((N, D), jnp.float32),
    )(x, w, b)


def _mm_scale(degp, t):
    return pl.pallas_call(
        _mm_scale_body,
        grid=(N // _R,),
        in_specs=[
            pl.BlockSpec((2, _R, 1), lambda i: (0, i, 0)),
            pl.BlockSpec((_R, D), lambda i: (i, 0)),
        ],
        out_specs=[
            pl.BlockSpec((_R, DW), lambda i: (i, 0)),
            pl.BlockSpec((_R, D), lambda i: (i, 0)),
        ],
        out_shape=[
            jax.ShapeDtypeStruct((N, DW), jnp.float32),
            jax.ShapeDtypeStruct((N, D), jnp.float32),
        ],
    )(degp, t)


def _mm_mid(dinv, h, sp, w, b):
    return pl.pallas_call(
        _mm_mid_body,
        grid=(N // _R,),
        in_specs=[
            pl.BlockSpec((_R, DW), lambda i: (i, 0)),
            pl.BlockSpec((_R, D), lambda i: (i, 0)),
            pl.BlockSpec((2, _R, D), lambda i: (0, i, 0)),
            pl.BlockSpec((D, D), lambda i: (0, 0)),
            pl.BlockSpec((1, D), lambda i: (0, 0)),
        ],
        out_specs=[
            pl.BlockSpec((_R, D), lambda i: (i, 0)),
            pl.BlockSpec((_R, D), lambda i: (i, 0)),
        ],
        out_shape=[
            jax.ShapeDtypeStruct((N, D), jnp.float32),
            jax.ShapeDtypeStruct((N, D), jnp.float32),
        ],
    )(dinv, h, sp, w, b)


def _mm_last(dinv, h, sp, w, b):
    return pl.pallas_call(
        _mm_last_body,
        grid=(N // _R,),
        in_specs=[
            pl.BlockSpec((_R, DW), lambda i: (i, 0)),
            pl.BlockSpec((_R, D), lambda i: (i, 0)),
            pl.BlockSpec((2, _R, D), lambda i: (0, i, 0)),
            pl.BlockSpec((D, D), lambda i: (0, 0)),
            pl.BlockSpec((1, D), lambda i: (0, 0)),
        ],
        out_specs=pl.BlockSpec((_R, D), lambda i: (i, 0)),
        out_shape=jax.ShapeDtypeStruct((N, D), jnp.float32),
    )(dinv, h, sp, w, b)


def kernel(x, edge_index, W1, b1, W2, b2, W3, b3, Wh, bh):
    row_fl = edge_index[0]
    col_fl = edge_index[1]
    nmain = NW * CH * KC
    row2 = row_fl[:nmain].reshape(NW, CH, 1, KC)
    col2 = col_fl[:nmain].reshape(NW, CH, 1, KC)
    trow = row_fl[nmain:].reshape(NTAIL, 1, KC)
    tcol = col_fl[nmain:].reshape(NTAIL, 1, KC)
    col = col_fl.reshape(NW, NCHUNK, K)
    b1r = b1.reshape(1, D)
    b2r = b2.reshape(1, D)
    b3r = b3.reshape(1, D)
    bhr = bh.reshape(1, D)

    t1 = _mm_plain(x, W1, b1r)
    degp = _degree(col).reshape(NC, NP_)[:, :N].reshape(NC, N, 1)
    dinv, m = _mm_scale(degp, t1)
    h = x
    for w, b in ((W2, b2r), (W3, b3r)):
        sp = _scatter_rows(m, row2, col2, trow, tcol).reshape(NC, N, D)
        h, m = _mm_mid(dinv, h, sp, w, b)
    sp = _scatter_rows(m, row2, col2, trow, tcol).reshape(NC, N, D)
    return _mm_last(dinv, h, sp, Wh, bhr)
